# baseline (speedup 1.0000x reference)
import functools

import jax
import jax.numpy as jnp
from jax import lax
from jax.experimental import pallas as pl
from jax.experimental.pallas import tpu as pltpu
from jax.experimental.pallas import tpu_sc as plsc

N = 10000
E = 320000
D = 128
NPAD = 10240
NC = 2
NS = 16
CH = 128
EP = 327680
PAD = EP - E
ER = EP // CH
ERC = ER // NC
RPT = ERC // NS
NBUF = 4
TRASH = NPAD - 1

_mesh = plsc.VectorSubcoreMesh(core_axis_name="c", subcore_axis_name="s")


def _fill(ref, n, value):
    def body(i, _):
        ref[pl.ds(pl.multiple_of(i * 16, 16), 16)] = jnp.full((16,), value, jnp.float32)
        return 0
    lax.fori_loop(0, n // 16, body, 0)


_DEG_RPT = 2 * ER // (NC * NS)


@functools.partial(
    pl.kernel,
    out_type=jax.ShapeDtypeStruct((2 * NPAD,), jnp.float32),
    mesh=_mesh,
    scratch_types=[
        pltpu.VMEM_SHARED((NPAD,), jnp.float32),
        pltpu.VMEM((_DEG_RPT, CH), jnp.int32),
        pltpu.VMEM((CH,), jnp.float32),
        pltpu.VMEM((640,), jnp.float32),
    ] + [pltpu.SemaphoreType.DMA] * NBUF,
)
def _sc_degree(edges_hbm, out_hbm, acc, ebuf, ones, zb, *sems):
    c = lax.axis_index("c")
    s = lax.axis_index("s")
    _fill(ones, CH, 1.0)
    _fill(zb, 640, 0.0)
    pltpu.sync_copy(zb, acc.at[pl.ds(pl.multiple_of(s * 640, 8), 640)])

    r0 = pl.multiple_of((c * NS + s) * _DEG_RPT, 8)
    pltpu.sync_copy(edges_hbm.at[pl.ds(r0, _DEG_RPT), :], ebuf)
    plsc.subcore_barrier()

    def fire(j, b):
        pltpu.async_copy(ones, acc.at[ebuf.at[j]], sems[b], add=True)

    for b in range(NBUF):
        fire(b, b)

    def block(k, _):
        for b in range(NBUF):
            j = k * NBUF + b
            pltpu.make_async_copy(ones, acc.at[ebuf.at[j]], sems[b]).wait()
            @pl.when(j + NBUF < _DEG_RPT)
            def _():
                fire(j + NBUF, b)
        return 0
    lax.fori_loop(0, _DEG_RPT // NBUF, block, 0)

    plsc.subcore_barrier()
    off = pl.multiple_of(c * NPAD + s * 640, 8)
    pltpu.sync_copy(acc.at[pl.ds(pl.multiple_of(s * 640, 8), 640)],
                    out_hbm.at[pl.ds(off, 640)])


GRP = 40
NGRP = RPT // GRP


@functools.partial(
    pl.kernel,
    out_type=jax.ShapeDtypeStruct((2 * NPAD, D), jnp.float32),
    mesh=_mesh,
    scratch_types=[
        pltpu.VMEM_SHARED((NPAD, D), jnp.float32),
        pltpu.VMEM((GRP, CH), jnp.int32),
        pltpu.VMEM((GRP, CH), jnp.int32),
        pltpu.VMEM((2, CH, D), jnp.float32),
    ] + [pltpu.SemaphoreType.DMA] * 2
      + [pltpu.SemaphoreType.DMA] * 2,
)
def _sc_agg(t_hbm, srcR_hbm, dstR_hbm, out_hbm, acc, sbuf, dbuf, rows, *sems):
    gsem = sems[:2]
    ssem = sems[2:]
    c = lax.axis_index("c")
    s = lax.axis_index("s")

    zr = rows.at[0]
    def zfill(i, _):
        for j in range(D // 16):
            zr[i, pl.ds(j * 16, 16)] = jnp.zeros((16,), jnp.float32)
        return 0
    lax.fori_loop(0, CH, zfill, 0)
    for k in range(640 // CH):
        pltpu.sync_copy(rows.at[0],
                        acc.at[pl.ds(pl.multiple_of(s * 640 + k * CH, 8), CH), :])

    r0 = pl.multiple_of(c * ERC + s * RPT, 8)

    def gather(j, b):
        pltpu.async_copy(t_hbm.at[sbuf.at[j]], rows.at[b], gsem[b])

    def scatter(j, b):
        pltpu.async_copy(rows.at[b], acc.at[dbuf.at[j]], ssem[b], add=True)

    def gwait(j, b):
        pltpu.make_async_copy(t_hbm.at[sbuf.at[j]], rows.at[b], gsem[b]).wait()

    def swait(j, b):
        pltpu.make_async_copy(rows.at[b], acc.at[dbuf.at[j]], ssem[b]).wait()

    plsc.subcore_barrier()

    for g in range(NGRP):
        gb = pl.multiple_of(r0 + g * GRP, 8)
        pltpu.sync_copy(srcR_hbm.at[pl.ds(gb, GRP), :], sbuf)
        pltpu.sync_copy(dstR_hbm.at[pl.ds(gb, GRP), :], dbuf)
        gather(0, 0)
        gather(1, 1)

        def block(k, _):
            for b in range(2):
                j = k * 2 + b
                gwait(j, b)
                scatter(j, b)
                swait(j, b)
                @pl.when(j + 2 < GRP)
                def _():
                    gather(j + 2, b)
            return 0
        lax.fori_loop(0, GRP // 2, block, 0)

    plsc.subcore_barrier()
    off = pl.multiple_of(c * NPAD + s * 640, 8)
    pltpu.sync_copy(acc.at[pl.ds(pl.multiple_of(s * 640, 8), 640), :],
                    out_hbm.at[pl.ds(off, 640), :])


EPT = E // (NC * NS)
SNB = 6
SFULL = EPT // CH
STAIL = EPT - SFULL * CH


@functools.partial(
    pl.kernel,
    out_type=(jax.ShapeDtypeStruct((E,), jnp.float32),
              jax.ShapeDtypeStruct((E,), jnp.float32)),
    mesh=_mesh,
    scratch_types=[
        pltpu.VMEM((EPT,), jnp.int32),
        pltpu.VMEM((EPT,), jnp.int32),
        pltpu.VMEM((EPT,), jnp.int32),
        pltpu.VMEM((EPT,), jnp.int32),
        pltpu.VMEM((EPT,), jnp.float32),
        pltpu.VMEM((EPT,), jnp.float32),
        pltpu.VMEM((SNB, CH), jnp.float32),
        pltpu.VMEM((SNB, CH), jnp.float32),
        pltpu.VMEM((STAIL,), jnp.float32),
        pltpu.VMEM((STAIL,), jnp.float32),
    ] + [pltpu.SemaphoreType.DMA] * (2 * SNB + 2),
)
def _sc_scores(sa_hbm, sb_hbm, src_hbm, dst_hbm, nsrc_hbm, ndst_hbm,
               pos_hbm, neg_hbm, srcv, dstv, nsv, ndv,
               pob, neb, ga, gb, ga_t, gb_t, *sems):
    asem = sems[:SNB]
    bsem = sems[SNB:2 * SNB]
    tsem = sems[2 * SNB:]
    c = lax.axis_index("c")
    s = lax.axis_index("s")
    w = c * NS + s
    base = pl.multiple_of(w * EPT, 8)

    descs = (
        pltpu.async_copy(src_hbm.at[pl.ds(base, EPT)], srcv, asem[0]),
        pltpu.async_copy(dst_hbm.at[pl.ds(base, EPT)], dstv, asem[1]),
        pltpu.async_copy(nsrc_hbm.at[pl.ds(base, EPT)], nsv, asem[2]),
        pltpu.async_copy(ndst_hbm.at[pl.ds(base, EPT)], ndv, bsem[0]),
    )
    for d in descs:
        d.wait()

    def run(av, bv, ob):
        def fire(j, b):
            o = pl.ds(pl.multiple_of(j * CH, 8), CH)
            pltpu.async_copy(sa_hbm.at[av.at[o]], ga.at[b], asem[b])
            pltpu.async_copy(sb_hbm.at[bv.at[o]], gb.at[b], bsem[b])

        def drain(j, b):
            o = pl.ds(pl.multiple_of(j * CH, 8), CH)
            pltpu.make_async_copy(sa_hbm.at[av.at[o]], ga.at[b], asem[b]).wait()
            pltpu.make_async_copy(sb_hbm.at[bv.at[o]], gb.at[b], bsem[b]).wait()

        for b in range(SNB):
            fire(b, b)

        def block(k, _):
            for b in range(SNB):
                j = k * SNB + b
                drain(j, b)
                for m in range(CH // 16):
                    o = pl.ds(pl.multiple_of(j * CH + m * 16, 8), 16)
                    om = pl.ds(m * 16, 16)
                    ob[o] = ga[b, om] + gb[b, om]
                @pl.when(j + SNB < SFULL)
                def _():
                    fire(j + SNB, b)
            return 0
        lax.fori_loop(0, SFULL // SNB, block, 0)

        ot = pl.ds(pl.multiple_of(SFULL * CH, 8), STAIL)
        pltpu.async_copy(sa_hbm.at[av.at[ot]], ga_t, tsem[0]).wait()
        pltpu.async_copy(sb_hbm.at[bv.at[ot]], gb_t, tsem[1]).wait()
        ob[ot] = ga_t[...] + gb_t[...]

    run(srcv, dstv, pob)
    run(nsv, ndv, neb)

    pltpu.sync_copy(pob, pos_hbm.at[pl.ds(base, EPT)])
    pltpu.sync_copy(neb, neg_hbm.at[pl.ds(base, EPT)])


def _norms(cnt_ref):
    cnt = cnt_ref[...]
    nrm = jnp.where(cnt > 0.0, lax.rsqrt(cnt), 0.0)
    return nrm[:N, 0:1], nrm[:N, 1:2]


def _join(agg_ref):
    return agg_ref[0, :N, :] + agg_ref[1, :N, :]


def _tc0_body(cnt_ref, x_ref, t0_ref):
    ns, _ = _norms(cnt_ref)
    t0_ref[...] = x_ref[...] * ns


def _tc1_body(cnt_ref, agg_ref, w1_ref, b1_ref, t1_ref):
    ns, nd = _norms(cnt_ref)
    agg = _join(agg_ref) * nd
    h1 = jax.nn.relu(jnp.dot(agg, w1_ref[...],
                             preferred_element_type=jnp.float32) + b1_ref[...])
    t1_ref[...] = h1 * ns


def _tc2_body(cnt_ref, agg_ref, w2_ref, b2_ref, w3_ref, t2_ref):
    ns, nd = _norms(cnt_ref)
    agg = _join(agg_ref) * nd
    h2 = jax.nn.relu(jnp.dot(agg, w2_ref[...],
                             preferred_element_type=jnp.float32) + b2_ref[...])
    t2_ref[...] = jnp.dot(h2 * ns, w3_ref[...], preferred_element_type=jnp.float32)


def _tc3_body(cnt_ref, agg_ref, b3_ref, wa_ref, wb_ref, blp_ref,
              h3_ref, sa_ref, sb_ref):
    _, nd = _norms(cnt_ref)
    h3 = _join(agg_ref) * nd + b3_ref[...]
    h3_ref[...] = h3
    sa_ref[...] = jnp.dot(h3, wa_ref[...], preferred_element_type=jnp.float32) + blp_ref[...]
    sb_ref[...] = jnp.dot(h3, wb_ref[...], preferred_element_type=jnp.float32)


_f32 = jnp.float32
_tshape = jax.ShapeDtypeStruct((N, D), _f32)

_tc0 = pl.pallas_call(_tc0_body, out_shape=_tshape)
_tc1 = pl.pallas_call(_tc1_body, out_shape=_tshape)
_tc2 = pl.pallas_call(_tc2_body, out_shape=_tshape)
_tc3 = pl.pallas_call(
    _tc3_body,
    out_shape=(jax.ShapeDtypeStruct((N, D), _f32),
               jax.ShapeDtypeStruct((N, 1), _f32),
               jax.ShapeDtypeStruct((N, 1), _f32)),
)


def kernel(x, edge_index, neg_src, neg_dst, W1, b1, W2, b2, W3, b3, Wlp, blp):
    src = edge_index[0]
    dst = edge_index[1]
    padt = jnp.full((PAD,), TRASH, jnp.int32)
    srcR = jnp.concatenate([src, jnp.zeros((PAD,), jnp.int32)]).reshape(ER, CH)
    dstR = jnp.concatenate([dst, padt]).reshape(ER, CH)

    edgesR = jnp.concatenate([src, padt, dst, padt]).reshape(2 * ER, CH)
    counts = _sc_degree(edgesR)
    cnt = counts.reshape(2, NPAD).T

    t0 = _tc0(cnt, x)
    agg1 = _sc_agg(t0, srcR, dstR).reshape(2, NPAD, D)
    t1 = _tc1(cnt, agg1, W1, b1.reshape(1, D))
    agg2 = _sc_agg(t1, srcR, dstR).reshape(2, NPAD, D)
    t2 = _tc2(cnt, agg2, W2, b2.reshape(1, 2 * D), W3)
    agg3 = _sc_agg(t2, srcR, dstR).reshape(2, NPAD, D)
    h3, sa, sb = _tc3(cnt, agg3, b3.reshape(1, D),
                      Wlp[:D], Wlp[D:], blp.reshape(1, 1))

    pos, neg = _sc_scores(sa.reshape(N), sb.reshape(N),
                          src, dst, neg_src, neg_dst)
    return (h3, pos, neg)
